# baseline (speedup 1.0000x reference)
import jax
import jax.numpy as jnp
from jax import lax
from jax.experimental import pallas as pl

BS = 1024
NUM_NODES = 500
SEQ_LEN = 24
K = 128
AUG_NUMS = 2
HIDDEN = 32


def _body(mapc_ref, mapr_ref, x_ref, y_ref, mask_ref, gum_ref, w1_ref,
          b1_ref, w2_ref, b2_ref, out_ref):
    m_col = mapc_ref[0, :, :]
    node_iota = lax.broadcasted_iota(jnp.int32, (K, NUM_NODES), 1)
    onehot = (m_col == node_iota).astype(jnp.float32)
    yblk = y_ref[0, :, :]
    smp = jnp.dot(onehot, yblk, preferred_element_type=jnp.float32)

    h = jnp.maximum(
        jnp.dot(smp, w1_ref[...], preferred_element_type=jnp.float32)
        + b1_ref[...], 0.0)
    out = jnp.dot(h, w2_ref[...], preferred_element_type=jnp.float32) \
        + b2_ref[...]
    g = -jnp.log(-jnp.log(gum_ref[0, :, :]))
    logits = out + g
    sel = logits[:, 1:2] > logits[:, 0:1]

    maskf = mask_ref[...] > 0.1
    val = jnp.where(sel & maskf, smp, 0.0)

    ji = lax.broadcasted_iota(jnp.int32, (K, K), 0)
    jj = lax.broadcasted_iota(jnp.int32, (K, K), 1)
    eq = m_col == mapr_ref[0, :, :]
    dup = jnp.any(eq & (jj > ji), axis=1, keepdims=True)
    winner = jnp.logical_not(dup).astype(jnp.float32)
    e = onehot * winner
    scat = lax.dot_general(e, val, (((0,), (0,)), ((), ())),
                           preferred_element_type=jnp.float32)
    ones = jnp.ones((K, 1), jnp.float32)
    hit = lax.dot_general(e, ones, (((0,), (0,)), ((), ())),
                          preferred_element_type=jnp.float32)
    out_ref[0, :, :] = jnp.where(hit > 0.0, scat, x_ref[0, :, :])


def kernel(x, y, map_id, idx_of_node, mask_u, gumbel_u, W1, b1, W2, b2):
    del idx_of_node
    mapc = map_id.reshape(BS, K, 1)
    mapr = map_id.reshape(BS, 1, K)
    gum3 = gumbel_u.reshape(BS, K, AUG_NUMS)
    b1r = b1.reshape(1, HIDDEN)
    b2r = b2.reshape(1, AUG_NUMS)
    x3 = x.reshape(BS, NUM_NODES, SEQ_LEN)
    y3 = y.reshape(BS, NUM_NODES, SEQ_LEN)
    out = pl.pallas_call(
        _body,
        grid=(BS,),
        in_specs=[
            pl.BlockSpec((1, K, 1), lambda b: (b, 0, 0)),
            pl.BlockSpec((1, 1, K), lambda b: (b, 0, 0)),
            pl.BlockSpec((1, NUM_NODES, SEQ_LEN), lambda b: (b, 0, 0)),
            pl.BlockSpec((1, NUM_NODES, SEQ_LEN), lambda b: (b, 0, 0)),
            pl.BlockSpec((K, SEQ_LEN), lambda b: (b, 0)),
            pl.BlockSpec((1, K, AUG_NUMS), lambda b: (b, 0, 0)),
            pl.BlockSpec((SEQ_LEN, HIDDEN), lambda b: (0, 0)),
            pl.BlockSpec((1, HIDDEN), lambda b: (0, 0)),
            pl.BlockSpec((HIDDEN, AUG_NUMS), lambda b: (0, 0)),
            pl.BlockSpec((1, AUG_NUMS), lambda b: (0, 0)),
        ],
        out_specs=pl.BlockSpec((1, NUM_NODES, SEQ_LEN), lambda b: (b, 0, 0)),
        out_shape=jax.ShapeDtypeStruct((BS, NUM_NODES, SEQ_LEN),
                                       jnp.float32),
    )(mapc, mapr, x3, y3, mask_u, gum3, W1, b1r, W2, b2r)
    return out.reshape(BS * NUM_NODES, SEQ_LEN)


# speedup vs baseline: 1.2748x; 1.2748x over previous
import jax
import jax.numpy as jnp
from jax import lax
from jax.experimental import pallas as pl

BS = 1024
NUM_NODES = 500
SEQ_LEN = 24
K = 128
AUG_NUMS = 2
HIDDEN = 32


G = 8


def _body(mapc_ref, mapr_ref, x_ref, y_ref, mask_ref, gum_ref, w1_ref,
          b1_ref, w2_ref, b2_ref, out_ref):
    for gi in range(G):
        m_col = mapc_ref[gi, :, :]
        node_iota = lax.broadcasted_iota(jnp.int32, (K, NUM_NODES), 1)
        onehot = (m_col == node_iota).astype(jnp.float32)
        yblk = y_ref[gi, :, :]
        smp = jnp.dot(onehot, yblk,
                      preferred_element_type=jnp.float32)

        h = jnp.maximum(
            jnp.dot(smp, w1_ref[...], preferred_element_type=jnp.float32)
            + b1_ref[...], 0.0)
        out = jnp.dot(h, w2_ref[...], preferred_element_type=jnp.float32) \
            + b2_ref[...]
        g = -jnp.log(-jnp.log(gum_ref[gi, :, :]))
        logits = out + g
        sel = logits[:, 1:2] > logits[:, 0:1]

        maskf = mask_ref[gi, :, :] > 0.1
        val = jnp.where(sel & maskf, smp, 0.0)

        ji = lax.broadcasted_iota(jnp.int32, (K, K), 0)
        jj = lax.broadcasted_iota(jnp.int32, (K, K), 1)
        eq = m_col == mapr_ref[gi, :, :]
        dup = jnp.any(eq & (jj > ji), axis=1, keepdims=True)
        winner = jnp.logical_not(dup).astype(jnp.float32)
        e = onehot * winner
        scat = lax.dot_general(e, val, (((0,), (0,)), ((), ())),
                               preferred_element_type=jnp.float32)
        ones = jnp.ones((K, 1), jnp.float32)
        hit = lax.dot_general(e, ones, (((0,), (0,)), ((), ())),
                              preferred_element_type=jnp.float32)
        out_ref[gi, :, :] = jnp.where(hit > 0.0, scat, x_ref[gi, :, :])


def kernel(x, y, map_id, idx_of_node, mask_u, gumbel_u, W1, b1, W2, b2):
    del idx_of_node
    mapc = map_id.reshape(BS, K, 1)
    mapr = map_id.reshape(BS, 1, K)
    gum3 = gumbel_u.reshape(BS, K, AUG_NUMS)
    b1r = b1.reshape(1, HIDDEN)
    b2r = b2.reshape(1, AUG_NUMS)
    x3 = x.reshape(BS, NUM_NODES, SEQ_LEN)
    y3 = y.reshape(BS, NUM_NODES, SEQ_LEN)
    mask3 = mask_u.reshape(BS, K, SEQ_LEN)
    out = pl.pallas_call(
        _body,
        grid=(BS // G,),
        in_specs=[
            pl.BlockSpec((G, K, 1), lambda b: (b, 0, 0)),
            pl.BlockSpec((G, 1, K), lambda b: (b, 0, 0)),
            pl.BlockSpec((G, NUM_NODES, SEQ_LEN), lambda b: (b, 0, 0)),
            pl.BlockSpec((G, NUM_NODES, SEQ_LEN), lambda b: (b, 0, 0)),
            pl.BlockSpec((G, K, SEQ_LEN), lambda b: (b, 0, 0)),
            pl.BlockSpec((G, K, AUG_NUMS), lambda b: (b, 0, 0)),
            pl.BlockSpec((SEQ_LEN, HIDDEN), lambda b: (0, 0)),
            pl.BlockSpec((1, HIDDEN), lambda b: (0, 0)),
            pl.BlockSpec((HIDDEN, AUG_NUMS), lambda b: (0, 0)),
            pl.BlockSpec((1, AUG_NUMS), lambda b: (0, 0)),
        ],
        out_specs=pl.BlockSpec((G, NUM_NODES, SEQ_LEN), lambda b: (b, 0, 0)),
        out_shape=jax.ShapeDtypeStruct((BS, NUM_NODES, SEQ_LEN),
                                       jnp.float32),
    )(mapc, mapr, x3, y3, mask3, gum3, W1, b1r, W2, b2r)
    return out.reshape(BS * NUM_NODES, SEQ_LEN)


# speedup vs baseline: 2.0231x; 1.5870x over previous
import jax
import jax.numpy as jnp
from jax import lax
from jax.experimental import pallas as pl

BS = 1024
NUM_NODES = 500
SEQ_LEN = 24
K = 128
AUG_NUMS = 2
HIDDEN = 32
G = 8


def _body(map_ref, g0_ref, g1_ref, x_ref, y_ref, mask_ref, w1_ref,
          b1_ref, w2_ref, b2_ref, out_ref):
    mapg = map_ref[...]
    t_blk = -jnp.log(-jnp.log(g0_ref[...])) \
        + jnp.log(-jnp.log(g1_ref[...]))
    eye = (lax.broadcasted_iota(jnp.int32, (K, K), 0)
           == lax.broadcasted_iota(jnp.int32, (K, K), 1)).astype(jnp.float32)
    tri = (lax.broadcasted_iota(jnp.int32, (K, K), 0)
           > lax.broadcasted_iota(jnp.int32, (K, K), 1)).astype(jnp.float32)
    node_iota = lax.broadcasted_iota(jnp.int32, (NUM_NODES, K), 0)

    for gi in range(G):
        m_row = mapg[gi:gi + 1, :]
        onehot_t = (node_iota == m_row).astype(jnp.float32)
        yblk = y_ref[gi * NUM_NODES:(gi + 1) * NUM_NODES, :]
        smp = lax.dot_general(onehot_t, yblk, (((0,), (0,)), ((), ())),
                              preferred_element_type=jnp.float32)

        h = jnp.maximum(
            jnp.dot(smp, w1_ref[...], preferred_element_type=jnp.float32)
            + b1_ref[...], 0.0)
        out = jnp.dot(h, w2_ref[...], preferred_element_type=jnp.float32) \
            + b2_ref[...]
        d_col = out[:, 1:2] - out[:, 0:1]
        t_row = t_blk[gi:gi + 1, :]
        t_col = lax.dot_general(eye, t_row, (((1,), (1,)), ((), ())),
                                preferred_element_type=jnp.float32)
        sel = d_col > t_col

        maskf = mask_ref[gi * K:(gi + 1) * K, :] > 0.1
        val = jnp.where(sel & maskf, smp, 0.0)
        ones_col = jnp.ones((K, 1), jnp.float32)
        val_p = jnp.concatenate([val, ones_col], axis=1)

        eq = lax.dot_general(onehot_t, onehot_t, (((0,), (0,)), ((), ())),
                             preferred_element_type=jnp.float32)
        loser = jnp.max(eq * tri, axis=0, keepdims=True)
        ew = onehot_t * (1.0 - loser)
        scat_p = jnp.dot(ew, val_p,
                         preferred_element_type=jnp.float32)
        hit = scat_p[:, SEQ_LEN:SEQ_LEN + 1]
        xblk = x_ref[gi * NUM_NODES:(gi + 1) * NUM_NODES, :]
        out_ref[gi * NUM_NODES:(gi + 1) * NUM_NODES, :] = jnp.where(
            hit > 0.0, scat_p[:, :SEQ_LEN], xblk)


def kernel(x, y, map_id, idx_of_node, mask_u, gumbel_u, W1, b1, W2, b2):
    del idx_of_node
    g2 = gumbel_u.reshape(BS, K, AUG_NUMS)
    g0 = g2[:, :, 0]
    g1 = g2[:, :, 1]
    b1r = b1.reshape(1, HIDDEN)
    b2r = b2.reshape(1, AUG_NUMS)
    return pl.pallas_call(
        _body,
        grid=(BS // G,),
        in_specs=[
            pl.BlockSpec((G, K), lambda b: (b, 0)),
            pl.BlockSpec((G, K), lambda b: (b, 0)),
            pl.BlockSpec((G, K), lambda b: (b, 0)),
            pl.BlockSpec((G * NUM_NODES, SEQ_LEN), lambda b: (b, 0)),
            pl.BlockSpec((G * NUM_NODES, SEQ_LEN), lambda b: (b, 0)),
            pl.BlockSpec((G * K, SEQ_LEN), lambda b: (b, 0)),
            pl.BlockSpec((SEQ_LEN, HIDDEN), lambda b: (0, 0)),
            pl.BlockSpec((1, HIDDEN), lambda b: (0, 0)),
            pl.BlockSpec((HIDDEN, AUG_NUMS), lambda b: (0, 0)),
            pl.BlockSpec((1, AUG_NUMS), lambda b: (0, 0)),
        ],
        out_specs=pl.BlockSpec((G * NUM_NODES, SEQ_LEN), lambda b: (b, 0)),
        out_shape=jax.ShapeDtypeStruct((BS * NUM_NODES, SEQ_LEN),
                                       jnp.float32),
    )(map_id, g0, g1, x, y, mask_u, W1, b1r, W2, b2r)


# speedup vs baseline: 2.1415x; 1.0586x over previous
import jax
import jax.numpy as jnp
from jax import lax
from jax.experimental import pallas as pl

BS = 1024
NUM_NODES = 500
SEQ_LEN = 24
K = 128
AUG_NUMS = 2
HIDDEN = 32
G = 16


def _body(map_ref, g0_ref, g1_ref, x_ref, y_ref, mask_ref, w1_ref,
          b1_ref, w2_ref, b2_ref, out_ref):
    mapg = map_ref[...]
    t_blk = -jnp.log(-jnp.log(g0_ref[...])) \
        + jnp.log(-jnp.log(g1_ref[...]))
    eye = (lax.broadcasted_iota(jnp.int32, (K, K), 0)
           == lax.broadcasted_iota(jnp.int32, (K, K), 1)).astype(jnp.float32)
    tri = (lax.broadcasted_iota(jnp.int32, (K, K), 0)
           > lax.broadcasted_iota(jnp.int32, (K, K), 1)).astype(jnp.float32)
    node_iota = lax.broadcasted_iota(jnp.int32, (NUM_NODES, K), 0)

    for gi in range(G):
        m_row = mapg[gi:gi + 1, :]
        onehot_t = (node_iota == m_row).astype(jnp.float32)
        yblk = y_ref[gi * NUM_NODES:(gi + 1) * NUM_NODES, :]
        smp = lax.dot_general(onehot_t, yblk, (((0,), (0,)), ((), ())),
                              preferred_element_type=jnp.float32)

        h = jnp.maximum(
            jnp.dot(smp, w1_ref[...], preferred_element_type=jnp.float32)
            + b1_ref[...], 0.0)
        out = jnp.dot(h, w2_ref[...], preferred_element_type=jnp.float32) \
            + b2_ref[...]
        d_col = out[:, 1:2] - out[:, 0:1]
        t_row = t_blk[gi:gi + 1, :]
        t_col = lax.dot_general(eye, t_row, (((1,), (1,)), ((), ())),
                                preferred_element_type=jnp.float32)
        sel = d_col > t_col

        maskf = mask_ref[gi * K:(gi + 1) * K, :] > 0.1
        val = jnp.where(sel & maskf, smp, 0.0)
        ones_col = jnp.ones((K, 1), jnp.float32)
        val_p = jnp.concatenate([val, ones_col], axis=1)

        eq = lax.dot_general(onehot_t, onehot_t, (((0,), (0,)), ((), ())),
                             preferred_element_type=jnp.float32)
        loser = jnp.max(eq * tri, axis=0, keepdims=True)
        ew = onehot_t * (1.0 - loser)
        scat_p = jnp.dot(ew, val_p,
                         preferred_element_type=jnp.float32)
        hit = scat_p[:, SEQ_LEN:SEQ_LEN + 1]
        xblk = x_ref[gi * NUM_NODES:(gi + 1) * NUM_NODES, :]
        out_ref[gi * NUM_NODES:(gi + 1) * NUM_NODES, :] = jnp.where(
            hit > 0.0, scat_p[:, :SEQ_LEN], xblk)


def kernel(x, y, map_id, idx_of_node, mask_u, gumbel_u, W1, b1, W2, b2):
    del idx_of_node
    g2 = gumbel_u.reshape(BS, K, AUG_NUMS)
    g0 = g2[:, :, 0]
    g1 = g2[:, :, 1]
    b1r = b1.reshape(1, HIDDEN)
    b2r = b2.reshape(1, AUG_NUMS)
    return pl.pallas_call(
        _body,
        grid=(BS // G,),
        in_specs=[
            pl.BlockSpec((G, K), lambda b: (b, 0)),
            pl.BlockSpec((G, K), lambda b: (b, 0)),
            pl.BlockSpec((G, K), lambda b: (b, 0)),
            pl.BlockSpec((G * NUM_NODES, SEQ_LEN), lambda b: (b, 0)),
            pl.BlockSpec((G * NUM_NODES, SEQ_LEN), lambda b: (b, 0)),
            pl.BlockSpec((G * K, SEQ_LEN), lambda b: (b, 0)),
            pl.BlockSpec((SEQ_LEN, HIDDEN), lambda b: (0, 0)),
            pl.BlockSpec((1, HIDDEN), lambda b: (0, 0)),
            pl.BlockSpec((HIDDEN, AUG_NUMS), lambda b: (0, 0)),
            pl.BlockSpec((1, AUG_NUMS), lambda b: (0, 0)),
        ],
        out_specs=pl.BlockSpec((G * NUM_NODES, SEQ_LEN), lambda b: (b, 0)),
        out_shape=jax.ShapeDtypeStruct((BS * NUM_NODES, SEQ_LEN),
                                       jnp.float32),
    )(map_id, g0, g1, x, y, mask_u, W1, b1r, W2, b2r)


# speedup vs baseline: 2.4396x; 1.1392x over previous
import jax
import jax.numpy as jnp
from jax import lax
from jax.experimental import pallas as pl

BS = 1024
NUM_NODES = 500
SEQ_LEN = 24
K = 128
AUG_NUMS = 2
HIDDEN = 32
G = 16


def _body(map_ref, g0_ref, g1_ref, x_ref, y_ref, mask_ref, w1t_ref,
          b1_ref, w2t_ref, b2d_ref, out_ref):
    mapg = map_ref[...]
    t_blk = -jnp.log(-jnp.log(g0_ref[...])) \
        + jnp.log(-jnp.log(g1_ref[...]))
    tri = (lax.broadcasted_iota(jnp.int32, (K, K), 0)
           > lax.broadcasted_iota(jnp.int32, (K, K), 1)).astype(jnp.float32)
    node_iota = lax.broadcasted_iota(jnp.int32, (NUM_NODES, K), 0)
    one_bf = jnp.bfloat16(1.0)
    zero_bf = jnp.bfloat16(0.0)

    for gi in range(G):
        m_row = mapg[gi:gi + 1, :]
        oh_f = (node_iota == m_row).astype(jnp.float32)
        oh_bf = oh_f.astype(jnp.bfloat16)
        yblk = y_ref[gi * NUM_NODES:(gi + 1) * NUM_NODES, :]

        smp_t = lax.dot_general(yblk, oh_f, (((0,), (0,)), ((), ())),
                                preferred_element_type=jnp.float32)
        h_t = jnp.maximum(
            jnp.dot(w1t_ref[...], smp_t, preferred_element_type=jnp.float32)
            + b1_ref[...], 0.0)
        out_t = jnp.dot(w2t_ref[...], h_t,
                        preferred_element_type=jnp.float32)
        d_row = out_t[1:2, :] - out_t[0:1, :] + b2d_ref[...]
        sel_row = jnp.where(d_row > t_blk[gi:gi + 1, :], 1.0,
                            0.0).astype(jnp.bfloat16)

        eq = lax.dot_general(oh_bf, oh_bf, (((0,), (0,)), ((), ())),
                             preferred_element_type=jnp.float32)
        loser = jnp.max(eq * tri, axis=0, keepdims=True)
        ew = oh_bf * (one_bf - loser.astype(jnp.bfloat16))

        m01 = jnp.where(mask_ref[gi * K:(gi + 1) * K, :] > 0.1,
                        1.0, 0.0).astype(jnp.bfloat16)
        p = ew * sel_row
        w = jnp.dot(p, m01, preferred_element_type=jnp.float32)
        hit = jnp.max(ew.astype(jnp.float32), axis=1, keepdims=True)

        xblk = x_ref[gi * NUM_NODES:(gi + 1) * NUM_NODES, :]
        out_ref[gi * NUM_NODES:(gi + 1) * NUM_NODES, :] = jnp.where(
            hit > 0.0, w * yblk, xblk)


def kernel(x, y, map_id, idx_of_node, mask_u, gumbel_u, W1, b1, W2, b2):
    del idx_of_node
    g2 = gumbel_u.reshape(BS, K, AUG_NUMS)
    g0 = g2[:, :, 0]
    g1 = g2[:, :, 1]
    w1t = W1.T.reshape(HIDDEN, SEQ_LEN)
    b1c = b1.reshape(HIDDEN, 1)
    w2t = W2.T.reshape(AUG_NUMS, HIDDEN)
    b2d = (b2[1] - b2[0]).reshape(1, 1)
    return pl.pallas_call(
        _body,
        grid=(BS // G,),
        in_specs=[
            pl.BlockSpec((G, K), lambda b: (b, 0)),
            pl.BlockSpec((G, K), lambda b: (b, 0)),
            pl.BlockSpec((G, K), lambda b: (b, 0)),
            pl.BlockSpec((G * NUM_NODES, SEQ_LEN), lambda b: (b, 0)),
            pl.BlockSpec((G * NUM_NODES, SEQ_LEN), lambda b: (b, 0)),
            pl.BlockSpec((G * K, SEQ_LEN), lambda b: (b, 0)),
            pl.BlockSpec((HIDDEN, SEQ_LEN), lambda b: (0, 0)),
            pl.BlockSpec((HIDDEN, 1), lambda b: (0, 0)),
            pl.BlockSpec((AUG_NUMS, HIDDEN), lambda b: (0, 0)),
            pl.BlockSpec((1, 1), lambda b: (0, 0)),
        ],
        out_specs=pl.BlockSpec((G * NUM_NODES, SEQ_LEN), lambda b: (b, 0)),
        out_shape=jax.ShapeDtypeStruct((BS * NUM_NODES, SEQ_LEN),
                                       jnp.float32),
    )(map_id, g0, g1, x, y, mask_u, w1t, b1c, w2t, b2d)
